# baseline (speedup 1.0000x reference)
import jax
import jax.numpy as jnp
from jax.experimental import pallas as pl
from jax.experimental.pallas import tpu as pltpu

NPAD = 128
BB = 512
K = 16


def _qnet_body(g_hbm, zsign_hbm, mask_hbm, out_hbm,
               vbuf, mbuf, zbuf, obuf, gsems, msem, zsem, osems):
    d = zbuf.shape[0]
    base = pl.program_id(0) * BB
    c = BB // K

    def gcopy(k):
        return pltpu.make_async_copy(
            g_hbm.at[pl.ds(base + k * c, c), 0, 0, :],
            vbuf.at[pl.ds(k * c, c), :],
            gsems.at[k])

    def ocopy(k):
        return pltpu.make_async_copy(
            obuf.at[pl.ds(k * c, c), :],
            out_hbm.at[pl.ds(base + k * c, c), 0, :],
            osems.at[k])

    zcopy = pltpu.make_async_copy(zsign_hbm, zbuf, zsem)
    mcopy = pltpu.make_async_copy(
        mask_hbm.at[pl.ds(base, BB), 0, :], mbuf, msem)

    zcopy.start()
    mcopy.start()
    for k in range(K):
        gcopy(k).start()
    zcopy.wait()
    mcopy.wait()
    z = zbuf[...]

    for k in range(K):
        gcopy(k).wait()
        v = vbuf[pl.ds(k * c, c), :]
        pr = v[:, :d]
        pi = v[:, d:]
        prob = pr * pr + pi * pi
        ev = jnp.dot(prob, z,
                     preferred_element_type=jnp.float32)
        obuf[pl.ds(k * c, c), :] = mbuf[pl.ds(k * c, c), :] * (ev + 1.0) * 0.5
        ocopy(k).start()

    for k in range(K):
        ocopy(k).wait()


def kernel(gates, zsign, mask):
    B, NG, D, D2 = gates.shape
    B_pad = -(-B // BB) * BB
    if B_pad != B:
        gates = jnp.pad(gates, ((0, B_pad - B), (0, 0), (0, 0), (0, 0)))
        mask = jnp.pad(mask, ((0, B_pad - B), (0, 0), (0, 0)))

    out = pl.pallas_call(
        _qnet_body,
        out_shape=jax.ShapeDtypeStruct((B_pad, 1, NPAD), jnp.float32),
        grid=(B_pad // BB,),
        in_specs=[
            pl.BlockSpec(memory_space=pl.ANY),
            pl.BlockSpec(memory_space=pl.ANY),
            pl.BlockSpec(memory_space=pl.ANY),
        ],
        out_specs=pl.BlockSpec(memory_space=pl.ANY),
        scratch_shapes=[
            pltpu.VMEM((BB, D2), jnp.float32),
            pltpu.VMEM((BB, NPAD), jnp.float32),
            pltpu.VMEM((D, NPAD), jnp.float32),
            pltpu.VMEM((BB, NPAD), jnp.float32),
            pltpu.SemaphoreType.DMA((K,)),
            pltpu.SemaphoreType.DMA,
            pltpu.SemaphoreType.DMA,
            pltpu.SemaphoreType.DMA((K,)),
        ],
        compiler_params=pltpu.CompilerParams(
            dimension_semantics=("parallel",)),
    )(gates, zsign, mask)
    return out[:B]


# speedup vs baseline: 1.8102x; 1.8102x over previous
import jax
import jax.numpy as jnp
from jax.experimental import pallas as pl
from jax.experimental.pallas import tpu as pltpu

NPAD = 128
BB = 512
K = 16


def _qnet_body(g_hbm, zsign_hbm, mask_hbm, out_hbm,
               vbuf, mbuf, zbuf, pbuf, obuf, gsems, msem, zsem, osems):
    d = zbuf.shape[0]
    base = pl.program_id(0) * BB
    c = BB // K

    def gcopy(k):
        return pltpu.make_async_copy(
            g_hbm.at[pl.ds(base + k * c, c), 0, 0, :],
            vbuf.at[pl.ds(k * c, c), :],
            gsems.at[k])

    zcopy = pltpu.make_async_copy(zsign_hbm, zbuf, zsem)
    mcopy = pltpu.make_async_copy(
        mask_hbm.at[pl.ds(base, BB), 0, :], mbuf, msem)

    zcopy.start()
    mcopy.start()
    for k in range(K):
        gcopy(k).start()
    zcopy.wait()
    mcopy.wait()
    z = zbuf[...]

    for k in range(K):
        gcopy(k).wait()
        v = vbuf[pl.ds(k * c, c), :]
        pr = v[:, :d]
        pi = v[:, d:]
        pbuf[pl.ds(k * c, c), :] = pr * pr + pi * pi

    ev = jnp.dot(pbuf[...], z,
                 preferred_element_type=jnp.float32)
    obuf[...] = mbuf[...] * (ev + 1.0) * 0.5
    ocopy = pltpu.make_async_copy(
        obuf, out_hbm.at[pl.ds(base, BB), 0, :], osems.at[0])
    ocopy.start()
    ocopy.wait()


def kernel(gates, zsign, mask):
    B, NG, D, D2 = gates.shape
    B_pad = -(-B // BB) * BB
    if B_pad != B:
        gates = jnp.pad(gates, ((0, B_pad - B), (0, 0), (0, 0), (0, 0)))
        mask = jnp.pad(mask, ((0, B_pad - B), (0, 0), (0, 0)))

    out = pl.pallas_call(
        _qnet_body,
        out_shape=jax.ShapeDtypeStruct((B_pad, 1, NPAD), jnp.float32),
        grid=(B_pad // BB,),
        in_specs=[
            pl.BlockSpec(memory_space=pl.ANY),
            pl.BlockSpec(memory_space=pl.ANY),
            pl.BlockSpec(memory_space=pl.ANY),
        ],
        out_specs=pl.BlockSpec(memory_space=pl.ANY),
        scratch_shapes=[
            pltpu.VMEM((BB, D2), jnp.float32),
            pltpu.VMEM((BB, NPAD), jnp.float32),
            pltpu.VMEM((D, NPAD), jnp.float32),
            pltpu.VMEM((BB, NPAD), jnp.float32),
            pltpu.VMEM((BB, NPAD), jnp.float32),
            pltpu.SemaphoreType.DMA((K,)),
            pltpu.SemaphoreType.DMA,
            pltpu.SemaphoreType.DMA,
            pltpu.SemaphoreType.DMA((1,)),
        ],
        compiler_params=pltpu.CompilerParams(
            dimension_semantics=("parallel",)),
    )(gates, zsign, mask)
    return out[:B]
